# baseline (speedup 1.0000x reference)
import jax
import jax.numpy as jnp
from jax import lax
from jax.experimental import pallas as pl
from jax.experimental.pallas import tpu as pltpu
from jax.experimental.pallas import tpu_sc as plsc

NN = 10000
EE = 320000
FD = 128
NC = 2
NS = 16
NW = NC * NS
EPW = EE // NW
NP = 10240
EPW2 = 10240
EP = NW * EPW2
CH = 32
NCHUNK = EPW2 // CH
IB = 16
NBLK = NCHUNK // IB
NBUF = 8
RPS = NP // NS
ZR = 16
NZCP = RPS // ZR


def _sc_mesh():
    return plsc.VectorSubcoreMesh(core_axis_name="c", subcore_axis_name="s",
                                  num_cores=NC, num_subcores=NS)


def _sc_degree(dstf):
    def body(dst_hbm, out_hbm, shared, dstv, deg, red, res):
        c = lax.axis_index("c")
        s = lax.axis_index("s")
        wid = c * NS + s
        pltpu.sync_copy(dst_hbm.at[wid], dstv)
        zero = jnp.zeros((16,), jnp.float32)

        def zstep(i, carry):
            deg[pl.ds(i * 16, 16)] = zero
            return carry

        lax.fori_loop(0, NP // 16, zstep, 0)

        def hstep(i, carry):
            idx = dstv[pl.ds(i * 16, 16)]
            cnt, last = plsc.scan_count(idx)
            plsc.addupdate_scatter(deg, [idx],
                                   cnt.astype(jnp.float32), mask=last)
            return carry

        lax.fori_loop(0, EPW // 16, hstep, 0)
        pltpu.sync_copy(deg, shared.at[s])
        plsc.subcore_barrier()
        for p in range(NS):
            pltpu.sync_copy(shared.at[p, pl.ds(s * RPS, RPS)], red.at[p])

        def rstep(k, carry):
            tot = red[0, pl.ds(k * 16, 16)]
            for p in range(1, NS):
                tot = tot + red[p, pl.ds(k * 16, 16)]
            res[pl.ds(k * 16, 16)] = tot
            return carry

        lax.fori_loop(0, RPS // 16, rstep, 0)
        pltpu.sync_copy(res, out_hbm.at[c, pl.ds(s * RPS, RPS)])

    f = pl.kernel(
        body,
        out_type=jax.ShapeDtypeStruct((NC, NP), jnp.float32),
        mesh=_sc_mesh(),
        compiler_params=pltpu.CompilerParams(needs_layout_passes=False),
        scratch_types=[
            pltpu.VMEM_SHARED((NS, NP), jnp.float32),
            pltpu.VMEM((EPW,), jnp.int32),
            pltpu.VMEM((NP,), jnp.float32),
            pltpu.VMEM((NS, RPS), jnp.float32),
            pltpu.VMEM((RPS,), jnp.float32),
        ],
    )
    return f(dstf)


def _sc_propagate(g, src4, dst4):
    def body(g_hbm, src_hbm, dst_hbm, out_hbm, acc,
             sia, dia, sib, dib, rows, zbuf,
             gsem, ssem, ixa, ixb):
        c = lax.axis_index("c")
        s = lax.axis_index("s")
        wid = c * NS + s
        pltpu.async_copy(src_hbm.at[wid, 0], sia, ixa)
        pltpu.async_copy(dst_hbm.at[wid, 0], dia, ixa)
        pltpu.async_copy(src_hbm.at[wid, 1], sib, ixb)
        pltpu.async_copy(dst_hbm.at[wid, 1], dib, ixb)
        zero = jnp.zeros((16,), jnp.float32)

        def zfill(i, carry):
            for k in range(FD // 16):
                zbuf[i, pl.ds(k * 16, 16)] = zero
            return carry

        lax.fori_loop(0, ZR, zfill, 0)
        for k in range(NZCP):
            pltpu.sync_copy(zbuf, acc.at[pl.ds(s * RPS + k * ZR, ZR)])
        plsc.subcore_barrier()

        def gfire(sblk, j, x):
            pltpu.async_copy(g_hbm.at[sblk.at[j]], rows[x], gsem[x])

        def gwait(sblk, j, x):
            pltpu.make_async_copy(g_hbm.at[sblk.at[j]], rows[x], gsem[x]).wait()

        def sfire(dblk, j, x):
            pltpu.async_copy(rows[x], acc.at[dblk.at[j]], ssem[x], add=True)

        def swait(dblk, j, x):
            pltpu.make_async_copy(rows[x], acc.at[dblk.at[j]], ssem[x]).wait()

        for b in range(NBLK):
            sblk, dblk, ixs = (sia, dia, ixa) if b % 2 == 0 else (sib, dib, ixb)
            pltpu.make_async_copy(src_hbm.at[wid, b], sblk, ixs).wait()
            pltpu.make_async_copy(dst_hbm.at[wid, b], dblk, ixs).wait()
            for x in range(NBUF):
                gfire(sblk, x, x)

            def quad(q, carry):
                j0 = NBUF * q
                for x in range(NBUF):
                    gwait(sblk, j0 + x, x)
                    sfire(dblk, j0 + x, x)
                for x in range(NBUF):
                    swait(dblk, j0 + x, x)

                    @pl.when(j0 + x + NBUF < IB)
                    def _():
                        gfire(sblk, j0 + x + NBUF, x)

                return carry

            lax.fori_loop(0, IB // NBUF, quad, 0)
            if b + 2 < NBLK:
                pltpu.async_copy(src_hbm.at[wid, b + 2], sblk, ixs)
                pltpu.async_copy(dst_hbm.at[wid, b + 2], dblk, ixs)
        plsc.subcore_barrier()
        pltpu.sync_copy(acc.at[pl.ds(s * RPS, RPS)],
                        out_hbm.at[c, pl.ds(s * RPS, RPS)])

    fn = pl.kernel(
        body,
        out_type=jax.ShapeDtypeStruct((NC, NP, FD), jnp.float32),
        mesh=_sc_mesh(),
        scratch_types=[
            pltpu.VMEM_SHARED((NP, FD), jnp.float32),
            pltpu.VMEM((IB, CH), jnp.int32),
            pltpu.VMEM((IB, CH), jnp.int32),
            pltpu.VMEM((IB, CH), jnp.int32),
            pltpu.VMEM((IB, CH), jnp.int32),
            [pltpu.VMEM((CH, FD), jnp.float32) for _ in range(NBUF)],
            pltpu.VMEM((ZR, FD), jnp.float32),
            [pltpu.SemaphoreType.DMA for _ in range(NBUF)],
            [pltpu.SemaphoreType.DMA for _ in range(NBUF)],
            pltpu.SemaphoreType.DMA,
            pltpu.SemaphoreType.DMA,
        ],
    )
    return fn(g, src4, dst4)


BN = 2000


def _tc_first(x, degp, w):
    def body(x_ref, dp_ref, w_ref, og_ref, dinv_ref):
        deg = dp_ref[0] + dp_ref[1] + 1.0
        dinv = lax.rsqrt(deg)
        dinv_ref[...] = dinv
        og_ref[...] = jnp.dot(x_ref[...], w_ref[...],
                              preferred_element_type=jnp.float32) * dinv

    return pl.pallas_call(
        body,
        grid=(NN // BN,),
        in_specs=[
            pl.BlockSpec((BN, FD), lambda i: (i, 0)),
            pl.BlockSpec((2, BN, 1), lambda i: (0, i, 0)),
            pl.BlockSpec((FD, FD), lambda i: (0, 0)),
        ],
        out_specs=[
            pl.BlockSpec((BN, FD), lambda i: (i, 0)),
            pl.BlockSpec((BN, 1), lambda i: (i, 0)),
        ],
        out_shape=[
            jax.ShapeDtypeStruct((NN, FD), jnp.float32),
            jax.ShapeDtypeStruct((NN, 1), jnp.float32),
        ],
    )(x, degp, w)


def _tc_layer(t, g, dinv, b, w, relu, emit_h=False):
    def body(t_ref, g_ref, dinv_ref, b_ref, w_ref, og_ref, *oh):
        h = (t_ref[0] + t_ref[1] + g_ref[...]) * dinv_ref[...] + b_ref[...]
        if relu:
            h = jnp.maximum(h, 0.0)
        if emit_h:
            oh[0][...] = h
        og_ref[...] = jnp.dot(h, w_ref[...],
                              preferred_element_type=jnp.float32) * dinv_ref[...]

    out_specs = [pl.BlockSpec((BN, FD), lambda i: (i, 0))]
    out_shape = [jax.ShapeDtypeStruct((NN, FD), jnp.float32)]
    if emit_h:
        out_specs.append(pl.BlockSpec((BN, FD), lambda i: (i, 0)))
        out_shape.append(jax.ShapeDtypeStruct((NN, FD), jnp.float32))
    res = pl.pallas_call(
        body,
        grid=(NN // BN,),
        in_specs=[
            pl.BlockSpec((2, BN, FD), lambda i: (0, i, 0)),
            pl.BlockSpec((BN, FD), lambda i: (i, 0)),
            pl.BlockSpec((BN, 1), lambda i: (i, 0)),
            pl.BlockSpec((1, FD), lambda i: (0, 0)),
            pl.BlockSpec((FD, FD), lambda i: (0, 0)),
        ],
        out_specs=out_specs,
        out_shape=out_shape,
    )(t, g, dinv, b, w)
    return res if emit_h else res[0]


def _tc_epilogue(t, g, dinv, b):
    def body(t_ref, g_ref, dinv_ref, b_ref, o_ref):
        o_ref[...] = ((t_ref[0] + t_ref[1] + g_ref[...]) * dinv_ref[...]
                      + b_ref[...])

    return pl.pallas_call(
        body,
        grid=(NN // BN,),
        in_specs=[
            pl.BlockSpec((2, BN, FD), lambda i: (0, i, 0)),
            pl.BlockSpec((BN, FD), lambda i: (i, 0)),
            pl.BlockSpec((BN, 1), lambda i: (i, 0)),
            pl.BlockSpec((1, FD), lambda i: (0, 0)),
        ],
        out_specs=pl.BlockSpec((BN, FD), lambda i: (i, 0)),
        out_shape=jax.ShapeDtypeStruct((NN, FD), jnp.float32),
    )(t, g, dinv, b)


def _pad_w(w):
    fi, fo = w.shape
    return jnp.pad(w, ((0, FD - fi), (0, FD - fo)))


def _pad_b(b):
    return jnp.pad(b, (0, FD - b.shape[0])).reshape(1, FD)


def kernel(x, edge_index, W1, b1, W2, b2, W3, b3, W4, b4, W5, b5, W6, b6):
    pad = EP - EE
    srcp = jnp.concatenate([edge_index[0],
                            jnp.arange(pad, dtype=jnp.int32) % NN])
    padrows = NN + (jnp.arange(pad, dtype=jnp.int32) % (NP - NN))
    dstp = jnp.concatenate([edge_index[1], padrows])
    src4 = srcp.reshape(NW, NBLK, IB, CH)
    dst4 = dstp.reshape(NW, NBLK, IB, CH)
    dstf = edge_index[1].reshape(NW, EPW)

    degp = _sc_degree(dstf)[:, :, None]
    g1, dinv = _tc_first(x, degp, _pad_w(W1))
    t1 = _sc_propagate(g1, src4, dst4)
    g2 = _tc_layer(t1, g1, dinv, _pad_b(b1), _pad_w(W2), relu=True)
    t2 = _sc_propagate(g2, src4, dst4)
    g3 = _tc_layer(t2, g2, dinv, _pad_b(b2), _pad_w(W3), relu=True)
    t3 = _sc_propagate(g3, src4, dst4)
    g4, z = _tc_layer(t3, g3, dinv, _pad_b(b3), _pad_w(W4),
                      relu=False, emit_h=True)
    t4 = _sc_propagate(g4, src4, dst4)
    g5 = _tc_layer(t4, g4, dinv, _pad_b(b4), _pad_w(W5), relu=True)
    t5 = _sc_propagate(g5, src4, dst4)
    g6 = _tc_layer(t5, g5, dinv, _pad_b(b5), _pad_w(W6), relu=True)
    t6 = _sc_propagate(g6, src4, dst4)
    x_recon = _tc_epilogue(t6, g6, dinv, _pad_b(b6))
    return (x_recon, z[:, :32])


# speedup vs baseline: 1.0773x; 1.0773x over previous
import jax
import jax.numpy as jnp
from jax import lax
from jax.experimental import pallas as pl
from jax.experimental.pallas import tpu as pltpu
from jax.experimental.pallas import tpu_sc as plsc

NN = 10000
EE = 320000
FD = 128
NC = 2
NS = 16
NW = NC * NS
EPW = EE // NW
NP = 10240
EPW2 = 10240
EP = NW * EPW2
CH = 64
NCHUNK = EPW2 // CH
IB = 16
NBLK = NCHUNK // IB
NBUF = 4
RPS = NP // NS
ZR = 16
NZCP = RPS // ZR


def _sc_mesh():
    return plsc.VectorSubcoreMesh(core_axis_name="c", subcore_axis_name="s",
                                  num_cores=NC, num_subcores=NS)


def _sc_degree(dstf):
    def body(dst_hbm, out_hbm, shared, dstv, deg, red, res):
        c = lax.axis_index("c")
        s = lax.axis_index("s")
        wid = c * NS + s
        pltpu.sync_copy(dst_hbm.at[wid], dstv)
        zero = jnp.zeros((16,), jnp.float32)

        def zstep(i, carry):
            deg[pl.ds(i * 16, 16)] = zero
            return carry

        lax.fori_loop(0, NP // 16, zstep, 0)

        def hstep(i, carry):
            idx = dstv[pl.ds(i * 16, 16)]
            cnt, last = plsc.scan_count(idx)
            plsc.addupdate_scatter(deg, [idx],
                                   cnt.astype(jnp.float32), mask=last)
            return carry

        lax.fori_loop(0, EPW // 16, hstep, 0)
        pltpu.sync_copy(deg, shared.at[s])
        plsc.subcore_barrier()
        for p in range(NS):
            pltpu.sync_copy(shared.at[p, pl.ds(s * RPS, RPS)], red.at[p])

        def rstep(k, carry):
            tot = red[0, pl.ds(k * 16, 16)]
            for p in range(1, NS):
                tot = tot + red[p, pl.ds(k * 16, 16)]
            res[pl.ds(k * 16, 16)] = tot
            return carry

        lax.fori_loop(0, RPS // 16, rstep, 0)
        pltpu.sync_copy(res, out_hbm.at[c, pl.ds(s * RPS, RPS)])

    f = pl.kernel(
        body,
        out_type=jax.ShapeDtypeStruct((NC, NP), jnp.float32),
        mesh=_sc_mesh(),
        compiler_params=pltpu.CompilerParams(needs_layout_passes=False),
        scratch_types=[
            pltpu.VMEM_SHARED((NS, NP), jnp.float32),
            pltpu.VMEM((EPW,), jnp.int32),
            pltpu.VMEM((NP,), jnp.float32),
            pltpu.VMEM((NS, RPS), jnp.float32),
            pltpu.VMEM((RPS,), jnp.float32),
        ],
    )
    return f(dstf)


def _sc_propagate(g, src4, dst4):
    def body(g_hbm, src_hbm, dst_hbm, out_hbm, acc,
             sia, dia, sib, dib, rows, zbuf,
             gsem, ssem, ixa, ixb):
        c = lax.axis_index("c")
        s = lax.axis_index("s")
        wid = c * NS + s
        pltpu.async_copy(src_hbm.at[wid, 0], sia, ixa)
        pltpu.async_copy(dst_hbm.at[wid, 0], dia, ixa)
        pltpu.async_copy(src_hbm.at[wid, 1], sib, ixb)
        pltpu.async_copy(dst_hbm.at[wid, 1], dib, ixb)
        zero = jnp.zeros((16,), jnp.float32)

        def zfill(i, carry):
            for k in range(FD // 16):
                zbuf[i, pl.ds(k * 16, 16)] = zero
            return carry

        lax.fori_loop(0, ZR, zfill, 0)
        for k in range(NZCP):
            pltpu.sync_copy(zbuf, acc.at[pl.ds(s * RPS + k * ZR, ZR)])
        plsc.subcore_barrier()

        def gfire(sblk, j, x):
            pltpu.async_copy(g_hbm.at[sblk.at[j]], rows[x], gsem[x])

        def gwait(sblk, j, x):
            pltpu.make_async_copy(g_hbm.at[sblk.at[j]], rows[x], gsem[x]).wait()

        def sfire(dblk, j, x):
            pltpu.async_copy(rows[x], acc.at[dblk.at[j]], ssem[x], add=True)

        def swait(dblk, j, x):
            pltpu.make_async_copy(rows[x], acc.at[dblk.at[j]], ssem[x]).wait()

        for b in range(NBLK):
            sblk, dblk, ixs = (sia, dia, ixa) if b % 2 == 0 else (sib, dib, ixb)
            pltpu.make_async_copy(src_hbm.at[wid, b], sblk, ixs).wait()
            pltpu.make_async_copy(dst_hbm.at[wid, b], dblk, ixs).wait()
            for x in range(NBUF):
                gfire(sblk, x, x)

            def quad(q, carry):
                j0 = NBUF * q
                for x in range(NBUF):
                    gwait(sblk, j0 + x, x)
                    sfire(dblk, j0 + x, x)
                for x in range(NBUF):
                    swait(dblk, j0 + x, x)

                    @pl.when(j0 + x + NBUF < IB)
                    def _():
                        gfire(sblk, j0 + x + NBUF, x)

                return carry

            lax.fori_loop(0, IB // NBUF, quad, 0)
            if b + 2 < NBLK:
                pltpu.async_copy(src_hbm.at[wid, b + 2], sblk, ixs)
                pltpu.async_copy(dst_hbm.at[wid, b + 2], dblk, ixs)
        plsc.subcore_barrier()
        pltpu.sync_copy(acc.at[pl.ds(s * RPS, RPS)],
                        out_hbm.at[c, pl.ds(s * RPS, RPS)])

    fn = pl.kernel(
        body,
        out_type=jax.ShapeDtypeStruct((NC, NP, FD), jnp.float32),
        mesh=_sc_mesh(),
        scratch_types=[
            pltpu.VMEM_SHARED((NP, FD), jnp.float32),
            pltpu.VMEM((IB, CH), jnp.int32),
            pltpu.VMEM((IB, CH), jnp.int32),
            pltpu.VMEM((IB, CH), jnp.int32),
            pltpu.VMEM((IB, CH), jnp.int32),
            [pltpu.VMEM((CH, FD), jnp.float32) for _ in range(NBUF)],
            pltpu.VMEM((ZR, FD), jnp.float32),
            [pltpu.SemaphoreType.DMA for _ in range(NBUF)],
            [pltpu.SemaphoreType.DMA for _ in range(NBUF)],
            pltpu.SemaphoreType.DMA,
            pltpu.SemaphoreType.DMA,
        ],
    )
    return fn(g, src4, dst4)


BN = 2000


def _tc_first(x, degp, w):
    def body(x_ref, dp_ref, w_ref, og_ref, dinv_ref):
        deg = dp_ref[0] + dp_ref[1] + 1.0
        dinv = lax.rsqrt(deg)
        dinv_ref[...] = dinv
        og_ref[...] = jnp.dot(x_ref[...], w_ref[...],
                              preferred_element_type=jnp.float32) * dinv

    return pl.pallas_call(
        body,
        grid=(NN // BN,),
        in_specs=[
            pl.BlockSpec((BN, FD), lambda i: (i, 0)),
            pl.BlockSpec((2, BN, 1), lambda i: (0, i, 0)),
            pl.BlockSpec((FD, FD), lambda i: (0, 0)),
        ],
        out_specs=[
            pl.BlockSpec((BN, FD), lambda i: (i, 0)),
            pl.BlockSpec((BN, 1), lambda i: (i, 0)),
        ],
        out_shape=[
            jax.ShapeDtypeStruct((NN, FD), jnp.float32),
            jax.ShapeDtypeStruct((NN, 1), jnp.float32),
        ],
    )(x, degp, w)


def _tc_layer(t, g, dinv, b, w, relu, emit_h=False):
    def body(t_ref, g_ref, dinv_ref, b_ref, w_ref, og_ref, *oh):
        h = (t_ref[0] + t_ref[1] + g_ref[...]) * dinv_ref[...] + b_ref[...]
        if relu:
            h = jnp.maximum(h, 0.0)
        if emit_h:
            oh[0][...] = h
        og_ref[...] = jnp.dot(h, w_ref[...],
                              preferred_element_type=jnp.float32) * dinv_ref[...]

    out_specs = [pl.BlockSpec((BN, FD), lambda i: (i, 0))]
    out_shape = [jax.ShapeDtypeStruct((NN, FD), jnp.float32)]
    if emit_h:
        out_specs.append(pl.BlockSpec((BN, FD), lambda i: (i, 0)))
        out_shape.append(jax.ShapeDtypeStruct((NN, FD), jnp.float32))
    res = pl.pallas_call(
        body,
        grid=(NN // BN,),
        in_specs=[
            pl.BlockSpec((2, BN, FD), lambda i: (0, i, 0)),
            pl.BlockSpec((BN, FD), lambda i: (i, 0)),
            pl.BlockSpec((BN, 1), lambda i: (i, 0)),
            pl.BlockSpec((1, FD), lambda i: (0, 0)),
            pl.BlockSpec((FD, FD), lambda i: (0, 0)),
        ],
        out_specs=out_specs,
        out_shape=out_shape,
    )(t, g, dinv, b, w)
    return res if emit_h else res[0]


def _tc_epilogue(t, g, dinv, b):
    def body(t_ref, g_ref, dinv_ref, b_ref, o_ref):
        o_ref[...] = ((t_ref[0] + t_ref[1] + g_ref[...]) * dinv_ref[...]
                      + b_ref[...])

    return pl.pallas_call(
        body,
        grid=(NN // BN,),
        in_specs=[
            pl.BlockSpec((2, BN, FD), lambda i: (0, i, 0)),
            pl.BlockSpec((BN, FD), lambda i: (i, 0)),
            pl.BlockSpec((BN, 1), lambda i: (i, 0)),
            pl.BlockSpec((1, FD), lambda i: (0, 0)),
        ],
        out_specs=pl.BlockSpec((BN, FD), lambda i: (i, 0)),
        out_shape=jax.ShapeDtypeStruct((NN, FD), jnp.float32),
    )(t, g, dinv, b)


def _pad_w(w):
    fi, fo = w.shape
    return jnp.pad(w, ((0, FD - fi), (0, FD - fo)))


def _pad_b(b):
    return jnp.pad(b, (0, FD - b.shape[0])).reshape(1, FD)


def kernel(x, edge_index, W1, b1, W2, b2, W3, b3, W4, b4, W5, b5, W6, b6):
    pad = EP - EE
    srcp = jnp.concatenate([edge_index[0],
                            jnp.arange(pad, dtype=jnp.int32) % NN])
    padrows = NN + (jnp.arange(pad, dtype=jnp.int32) % (NP - NN))
    dstp = jnp.concatenate([edge_index[1], padrows])
    src4 = srcp.reshape(NW, NBLK, IB, CH)
    dst4 = dstp.reshape(NW, NBLK, IB, CH)
    dstf = edge_index[1].reshape(NW, EPW)

    degp = _sc_degree(dstf)[:, :, None]
    g1, dinv = _tc_first(x, degp, _pad_w(W1))
    t1 = _sc_propagate(g1, src4, dst4)
    g2 = _tc_layer(t1, g1, dinv, _pad_b(b1), _pad_w(W2), relu=True)
    t2 = _sc_propagate(g2, src4, dst4)
    g3 = _tc_layer(t2, g2, dinv, _pad_b(b2), _pad_w(W3), relu=True)
    t3 = _sc_propagate(g3, src4, dst4)
    g4, z = _tc_layer(t3, g3, dinv, _pad_b(b3), _pad_w(W4),
                      relu=False, emit_h=True)
    t4 = _sc_propagate(g4, src4, dst4)
    g5 = _tc_layer(t4, g4, dinv, _pad_b(b4), _pad_w(W5), relu=True)
    t5 = _sc_propagate(g5, src4, dst4)
    g6 = _tc_layer(t5, g5, dinv, _pad_b(b5), _pad_w(W6), relu=True)
    t6 = _sc_propagate(g6, src4, dst4)
    x_recon = _tc_epilogue(t6, g6, dinv, _pad_b(b6))
    return (x_recon, z[:, :32])


# speedup vs baseline: 1.1380x; 1.0563x over previous
import jax
import jax.numpy as jnp
from jax import lax
from jax.experimental import pallas as pl
from jax.experimental.pallas import tpu as pltpu
from jax.experimental.pallas import tpu_sc as plsc

NN = 10000
EE = 320000
FD = 128
NC = 2
NS = 16
NW = NC * NS
EPW = EE // NW
NP = 10240
EPW2 = 10240
EP = NW * EPW2
CH = 64
NCHUNK = EPW2 // CH
IB = 16
NBLK = NCHUNK // IB
NBUF = 4
RPS = NP // NS
ZR = 32
NZCP = RPS // ZR


def _sc_mesh():
    return plsc.VectorSubcoreMesh(core_axis_name="c", subcore_axis_name="s",
                                  num_cores=NC, num_subcores=NS)


def _sc_degree(dstf):
    def body(dst_hbm, out_hbm, shared, dstv, deg, red, res):
        c = lax.axis_index("c")
        s = lax.axis_index("s")
        wid = c * NS + s
        pltpu.sync_copy(dst_hbm.at[wid], dstv)
        zero = jnp.zeros((16,), jnp.float32)

        def zstep(i, carry):
            deg[pl.ds(i * 16, 16)] = zero
            return carry

        lax.fori_loop(0, NP // 16, zstep, 0)

        def hstep(i, carry):
            idx = dstv[pl.ds(i * 16, 16)]
            cnt, last = plsc.scan_count(idx)
            plsc.addupdate_scatter(deg, [idx],
                                   cnt.astype(jnp.float32), mask=last)
            return carry

        lax.fori_loop(0, EPW // 16, hstep, 0)
        pltpu.sync_copy(deg, shared.at[s])
        plsc.subcore_barrier()
        for p in range(NS):
            pltpu.sync_copy(shared.at[p, pl.ds(s * RPS, RPS)], red.at[p])

        def rstep(k, carry):
            tot = red[0, pl.ds(k * 16, 16)]
            for p in range(1, NS):
                tot = tot + red[p, pl.ds(k * 16, 16)]
            res[pl.ds(k * 16, 16)] = tot
            return carry

        lax.fori_loop(0, RPS // 16, rstep, 0)
        pltpu.sync_copy(res, out_hbm.at[c, pl.ds(s * RPS, RPS)])

    f = pl.kernel(
        body,
        out_type=jax.ShapeDtypeStruct((NC, NP), jnp.float32),
        mesh=_sc_mesh(),
        compiler_params=pltpu.CompilerParams(needs_layout_passes=False),
        scratch_types=[
            pltpu.VMEM_SHARED((NS, NP), jnp.float32),
            pltpu.VMEM((EPW,), jnp.int32),
            pltpu.VMEM((NP,), jnp.float32),
            pltpu.VMEM((NS, RPS), jnp.float32),
            pltpu.VMEM((RPS,), jnp.float32),
        ],
    )
    return f(dstf)


def _sc_propagate(g, src4, dst4):
    def body(g_hbm, src_hbm, dst_hbm, out_hbm, acc,
             sia, dia, sib, dib, rows, zbuf,
             gsem, ssem, ixa, ixb):
        c = lax.axis_index("c")
        s = lax.axis_index("s")
        wid = c * NS + s
        pltpu.async_copy(src_hbm.at[wid, 0], sia, ixa)
        pltpu.async_copy(dst_hbm.at[wid, 0], dia, ixa)
        pltpu.async_copy(src_hbm.at[wid, 1], sib, ixb)
        pltpu.async_copy(dst_hbm.at[wid, 1], dib, ixb)
        zero = jnp.zeros((16,), jnp.float32)

        def zfill(i, carry):
            for k in range(FD // 16):
                zbuf[i, pl.ds(k * 16, 16)] = zero
            return carry

        lax.fori_loop(0, ZR, zfill, 0)
        for k in range(NZCP):
            pltpu.sync_copy(zbuf, acc.at[pl.ds(s * RPS + k * ZR, ZR)])
        plsc.subcore_barrier()

        def gfire(sblk, j, x):
            pltpu.async_copy(g_hbm.at[sblk.at[j]], rows[x], gsem[x])

        def gwait(sblk, j, x):
            pltpu.make_async_copy(g_hbm.at[sblk.at[j]], rows[x], gsem[x]).wait()

        def sfire(dblk, j, x):
            pltpu.async_copy(rows[x], acc.at[dblk.at[j]], ssem[x], add=True)

        def swait(dblk, j, x):
            pltpu.make_async_copy(rows[x], acc.at[dblk.at[j]], ssem[x]).wait()

        pltpu.make_async_copy(src_hbm.at[wid, 0], sia, ixa).wait()
        pltpu.make_async_copy(dst_hbm.at[wid, 0], dia, ixa).wait()
        for x in range(NBUF):
            gfire(sia, x, x)
        for b in range(NBLK):
            sblk, dblk, ixs = (sia, dia, ixa) if b % 2 == 0 else (sib, dib, ixb)
            nsblk, ndblk, nixs = (sia, dia, ixa) if b % 2 else (sib, dib, ixb)

            def quad(q, carry):
                j0 = NBUF * q
                for x in range(NBUF):
                    gwait(sblk, j0 + x, x)
                    sfire(dblk, j0 + x, x)
                for x in range(NBUF):
                    swait(dblk, j0 + x, x)
                    gfire(sblk, j0 + x + NBUF, x)
                return carry

            lax.fori_loop(0, IB // NBUF - 1, quad, 0)
            if b + 1 < NBLK:
                pltpu.make_async_copy(src_hbm.at[wid, b + 1], nsblk, nixs).wait()
                pltpu.make_async_copy(dst_hbm.at[wid, b + 1], ndblk, nixs).wait()
            for x in range(NBUF):
                gwait(sblk, IB - NBUF + x, x)
                sfire(dblk, IB - NBUF + x, x)
            for x in range(NBUF):
                swait(dblk, IB - NBUF + x, x)
                if b + 1 < NBLK:
                    gfire(nsblk, x, x)
            if b + 2 < NBLK:
                pltpu.async_copy(src_hbm.at[wid, b + 2], sblk, ixs)
                pltpu.async_copy(dst_hbm.at[wid, b + 2], dblk, ixs)
        plsc.subcore_barrier()
        pltpu.sync_copy(acc.at[pl.ds(s * RPS, RPS)],
                        out_hbm.at[c, pl.ds(s * RPS, RPS)])

    fn = pl.kernel(
        body,
        out_type=jax.ShapeDtypeStruct((NC, NP, FD), jnp.float32),
        mesh=_sc_mesh(),
        scratch_types=[
            pltpu.VMEM_SHARED((NP, FD), jnp.float32),
            pltpu.VMEM((IB, CH), jnp.int32),
            pltpu.VMEM((IB, CH), jnp.int32),
            pltpu.VMEM((IB, CH), jnp.int32),
            pltpu.VMEM((IB, CH), jnp.int32),
            [pltpu.VMEM((CH, FD), jnp.float32) for _ in range(NBUF)],
            pltpu.VMEM((ZR, FD), jnp.float32),
            [pltpu.SemaphoreType.DMA for _ in range(NBUF)],
            [pltpu.SemaphoreType.DMA for _ in range(NBUF)],
            pltpu.SemaphoreType.DMA,
            pltpu.SemaphoreType.DMA,
        ],
    )
    return fn(g, src4, dst4)


BN = 2000


def _tc_first(x, degp, w):
    def body(x_ref, dp_ref, w_ref, og_ref, dinv_ref):
        deg = dp_ref[0] + dp_ref[1] + 1.0
        dinv = lax.rsqrt(deg)
        dinv_ref[...] = dinv
        og_ref[...] = jnp.dot(x_ref[...], w_ref[...],
                              preferred_element_type=jnp.float32) * dinv

    return pl.pallas_call(
        body,
        grid=(NN // BN,),
        in_specs=[
            pl.BlockSpec((BN, FD), lambda i: (i, 0)),
            pl.BlockSpec((2, BN, 1), lambda i: (0, i, 0)),
            pl.BlockSpec((FD, FD), lambda i: (0, 0)),
        ],
        out_specs=[
            pl.BlockSpec((BN, FD), lambda i: (i, 0)),
            pl.BlockSpec((BN, 1), lambda i: (i, 0)),
        ],
        out_shape=[
            jax.ShapeDtypeStruct((NN, FD), jnp.float32),
            jax.ShapeDtypeStruct((NN, 1), jnp.float32),
        ],
    )(x, degp, w)


def _tc_layer(t, g, dinv, b, w, relu, emit_h=False):
    def body(t_ref, g_ref, dinv_ref, b_ref, w_ref, og_ref, *oh):
        h = (t_ref[0] + t_ref[1] + g_ref[...]) * dinv_ref[...] + b_ref[...]
        if relu:
            h = jnp.maximum(h, 0.0)
        if emit_h:
            oh[0][...] = h
        og_ref[...] = jnp.dot(h, w_ref[...],
                              preferred_element_type=jnp.float32) * dinv_ref[...]

    out_specs = [pl.BlockSpec((BN, FD), lambda i: (i, 0))]
    out_shape = [jax.ShapeDtypeStruct((NN, FD), jnp.float32)]
    if emit_h:
        out_specs.append(pl.BlockSpec((BN, FD), lambda i: (i, 0)))
        out_shape.append(jax.ShapeDtypeStruct((NN, FD), jnp.float32))
    res = pl.pallas_call(
        body,
        grid=(NN // BN,),
        in_specs=[
            pl.BlockSpec((2, BN, FD), lambda i: (0, i, 0)),
            pl.BlockSpec((BN, FD), lambda i: (i, 0)),
            pl.BlockSpec((BN, 1), lambda i: (i, 0)),
            pl.BlockSpec((1, FD), lambda i: (0, 0)),
            pl.BlockSpec((FD, FD), lambda i: (0, 0)),
        ],
        out_specs=out_specs,
        out_shape=out_shape,
    )(t, g, dinv, b, w)
    return res if emit_h else res[0]


def _tc_epilogue(t, g, dinv, b):
    def body(t_ref, g_ref, dinv_ref, b_ref, o_ref):
        o_ref[...] = ((t_ref[0] + t_ref[1] + g_ref[...]) * dinv_ref[...]
                      + b_ref[...])

    return pl.pallas_call(
        body,
        grid=(NN // BN,),
        in_specs=[
            pl.BlockSpec((2, BN, FD), lambda i: (0, i, 0)),
            pl.BlockSpec((BN, FD), lambda i: (i, 0)),
            pl.BlockSpec((BN, 1), lambda i: (i, 0)),
            pl.BlockSpec((1, FD), lambda i: (0, 0)),
        ],
        out_specs=pl.BlockSpec((BN, FD), lambda i: (i, 0)),
        out_shape=jax.ShapeDtypeStruct((NN, FD), jnp.float32),
    )(t, g, dinv, b)


def _pad_w(w):
    fi, fo = w.shape
    return jnp.pad(w, ((0, FD - fi), (0, FD - fo)))


def _pad_b(b):
    return jnp.pad(b, (0, FD - b.shape[0])).reshape(1, FD)


def kernel(x, edge_index, W1, b1, W2, b2, W3, b3, W4, b4, W5, b5, W6, b6):
    pad = EP - EE
    srcp = jnp.concatenate([edge_index[0],
                            jnp.arange(pad, dtype=jnp.int32) % NN])
    padrows = NN + (jnp.arange(pad, dtype=jnp.int32) % (NP - NN))
    dstp = jnp.concatenate([edge_index[1], padrows])
    src4 = srcp.reshape(NW, NBLK, IB, CH)
    dst4 = dstp.reshape(NW, NBLK, IB, CH)
    dstf = edge_index[1].reshape(NW, EPW)

    degp = _sc_degree(dstf)[:, :, None]
    g1, dinv = _tc_first(x, degp, _pad_w(W1))
    t1 = _sc_propagate(g1, src4, dst4)
    g2 = _tc_layer(t1, g1, dinv, _pad_b(b1), _pad_w(W2), relu=True)
    t2 = _sc_propagate(g2, src4, dst4)
    g3 = _tc_layer(t2, g2, dinv, _pad_b(b2), _pad_w(W3), relu=True)
    t3 = _sc_propagate(g3, src4, dst4)
    g4, z = _tc_layer(t3, g3, dinv, _pad_b(b3), _pad_w(W4),
                      relu=False, emit_h=True)
    t4 = _sc_propagate(g4, src4, dst4)
    g5 = _tc_layer(t4, g4, dinv, _pad_b(b4), _pad_w(W5), relu=True)
    t5 = _sc_propagate(g5, src4, dst4)
    g6 = _tc_layer(t5, g5, dinv, _pad_b(b5), _pad_w(W6), relu=True)
    t6 = _sc_propagate(g6, src4, dst4)
    x_recon = _tc_epilogue(t6, g6, dinv, _pad_b(b6))
    return (x_recon, z[:, :32])


# speedup vs baseline: 1.1418x; 1.0033x over previous
import jax
import jax.numpy as jnp
from jax import lax
from jax.experimental import pallas as pl
from jax.experimental.pallas import tpu as pltpu
from jax.experimental.pallas import tpu_sc as plsc

NN = 10000
EE = 320000
FD = 128
NC = 2
NS = 16
NW = NC * NS
EPW = EE // NW
NP = 10240
EPW2 = 10240
EP = NW * EPW2
CH = 80
NCHUNK = EPW2 // CH
IB = 8
NBLK = NCHUNK // IB
NBUF = 4
RPS = NP // NS
ZR = 16
NZCP = RPS // ZR


def _sc_mesh():
    return plsc.VectorSubcoreMesh(core_axis_name="c", subcore_axis_name="s",
                                  num_cores=NC, num_subcores=NS)


def _sc_degree(dstf):
    def body(dst_hbm, out_hbm, shared, dstv, deg, red, res):
        c = lax.axis_index("c")
        s = lax.axis_index("s")
        wid = c * NS + s
        pltpu.sync_copy(dst_hbm.at[wid], dstv)
        zero = jnp.zeros((16,), jnp.float32)

        def zstep(i, carry):
            deg[pl.ds(i * 16, 16)] = zero
            return carry

        lax.fori_loop(0, NP // 16, zstep, 0)

        def hstep(i, carry):
            idx = dstv[pl.ds(i * 16, 16)]
            cnt, last = plsc.scan_count(idx)
            plsc.addupdate_scatter(deg, [idx],
                                   cnt.astype(jnp.float32), mask=last)
            return carry

        lax.fori_loop(0, EPW // 16, hstep, 0)
        pltpu.sync_copy(deg, shared.at[s])
        plsc.subcore_barrier()
        for p in range(NS):
            pltpu.sync_copy(shared.at[p, pl.ds(s * RPS, RPS)], red.at[p])

        def rstep(k, carry):
            tot = red[0, pl.ds(k * 16, 16)]
            for p in range(1, NS):
                tot = tot + red[p, pl.ds(k * 16, 16)]
            res[pl.ds(k * 16, 16)] = tot
            return carry

        lax.fori_loop(0, RPS // 16, rstep, 0)
        pltpu.sync_copy(res, out_hbm.at[c, pl.ds(s * RPS, RPS)])

    f = pl.kernel(
        body,
        out_type=jax.ShapeDtypeStruct((NC, NP), jnp.float32),
        mesh=_sc_mesh(),
        compiler_params=pltpu.CompilerParams(needs_layout_passes=False),
        scratch_types=[
            pltpu.VMEM_SHARED((NS, NP), jnp.float32),
            pltpu.VMEM((EPW,), jnp.int32),
            pltpu.VMEM((NP,), jnp.float32),
            pltpu.VMEM((NS, RPS), jnp.float32),
            pltpu.VMEM((RPS,), jnp.float32),
        ],
    )
    return f(dstf)


def _sc_propagate(g, src4, dst4):
    def body(g_hbm, src_hbm, dst_hbm, out_hbm, acc,
             sia, dia, sib, dib, rows, zbuf,
             gsem, ssem, ixa, ixb):
        c = lax.axis_index("c")
        s = lax.axis_index("s")
        wid = c * NS + s
        pltpu.async_copy(src_hbm.at[wid, 0], sia, ixa)
        pltpu.async_copy(dst_hbm.at[wid, 0], dia, ixa)
        pltpu.async_copy(src_hbm.at[wid, 1], sib, ixb)
        pltpu.async_copy(dst_hbm.at[wid, 1], dib, ixb)
        zero = jnp.zeros((16,), jnp.float32)

        def zfill(i, carry):
            for k in range(FD // 16):
                zbuf[i, pl.ds(k * 16, 16)] = zero
            return carry

        lax.fori_loop(0, ZR, zfill, 0)
        for k in range(NZCP):
            pltpu.sync_copy(zbuf, acc.at[pl.ds(s * RPS + k * ZR, ZR)])
        plsc.subcore_barrier()

        def gfire(sblk, j, x):
            pltpu.async_copy(g_hbm.at[sblk.at[j]], rows[x], gsem[x])

        def gwait(sblk, j, x):
            pltpu.make_async_copy(g_hbm.at[sblk.at[j]], rows[x], gsem[x]).wait()

        def sfire(dblk, j, x):
            pltpu.async_copy(rows[x], acc.at[dblk.at[j]], ssem[x], add=True)

        def swait(dblk, j, x):
            pltpu.make_async_copy(rows[x], acc.at[dblk.at[j]], ssem[x]).wait()

        pltpu.make_async_copy(src_hbm.at[wid, 0], sia, ixa).wait()
        pltpu.make_async_copy(dst_hbm.at[wid, 0], dia, ixa).wait()
        for x in range(NBUF):
            gfire(sia, x, x)
        for b in range(NBLK):
            sblk, dblk, ixs = (sia, dia, ixa) if b % 2 == 0 else (sib, dib, ixb)
            nsblk, ndblk, nixs = (sia, dia, ixa) if b % 2 else (sib, dib, ixb)

            def quad(q, carry):
                j0 = NBUF * q
                for x in range(NBUF):
                    gwait(sblk, j0 + x, x)
                    sfire(dblk, j0 + x, x)
                for x in range(NBUF):
                    swait(dblk, j0 + x, x)
                    gfire(sblk, j0 + x + NBUF, x)
                return carry

            lax.fori_loop(0, IB // NBUF - 1, quad, 0)
            if b + 1 < NBLK:
                pltpu.make_async_copy(src_hbm.at[wid, b + 1], nsblk, nixs).wait()
                pltpu.make_async_copy(dst_hbm.at[wid, b + 1], ndblk, nixs).wait()
            for x in range(NBUF):
                gwait(sblk, IB - NBUF + x, x)
                sfire(dblk, IB - NBUF + x, x)
            for x in range(NBUF):
                swait(dblk, IB - NBUF + x, x)
                if b + 1 < NBLK:
                    gfire(nsblk, x, x)
            if b + 2 < NBLK:
                pltpu.async_copy(src_hbm.at[wid, b + 2], sblk, ixs)
                pltpu.async_copy(dst_hbm.at[wid, b + 2], dblk, ixs)
        plsc.subcore_barrier()
        pltpu.sync_copy(acc.at[pl.ds(s * RPS, RPS)],
                        out_hbm.at[c, pl.ds(s * RPS, RPS)])

    fn = pl.kernel(
        body,
        out_type=jax.ShapeDtypeStruct((NC, NP, FD), jnp.float32),
        mesh=_sc_mesh(),
        scratch_types=[
            pltpu.VMEM_SHARED((NP, FD), jnp.float32),
            pltpu.VMEM((IB, CH), jnp.int32),
            pltpu.VMEM((IB, CH), jnp.int32),
            pltpu.VMEM((IB, CH), jnp.int32),
            pltpu.VMEM((IB, CH), jnp.int32),
            [pltpu.VMEM((CH, FD), jnp.float32) for _ in range(NBUF)],
            pltpu.VMEM((ZR, FD), jnp.float32),
            [pltpu.SemaphoreType.DMA for _ in range(NBUF)],
            [pltpu.SemaphoreType.DMA for _ in range(NBUF)],
            pltpu.SemaphoreType.DMA,
            pltpu.SemaphoreType.DMA,
        ],
    )
    return fn(g, src4, dst4)


BN = 2000


def _tc_first(x, degp, w):
    def body(x_ref, dp_ref, w_ref, og_ref, dinv_ref):
        deg = dp_ref[0] + dp_ref[1] + 1.0
        dinv = lax.rsqrt(deg)
        dinv_ref[...] = dinv
        og_ref[...] = jnp.dot(x_ref[...], w_ref[...],
                              preferred_element_type=jnp.float32) * dinv

    return pl.pallas_call(
        body,
        grid=(NN // BN,),
        in_specs=[
            pl.BlockSpec((BN, FD), lambda i: (i, 0)),
            pl.BlockSpec((2, BN, 1), lambda i: (0, i, 0)),
            pl.BlockSpec((FD, FD), lambda i: (0, 0)),
        ],
        out_specs=[
            pl.BlockSpec((BN, FD), lambda i: (i, 0)),
            pl.BlockSpec((BN, 1), lambda i: (i, 0)),
        ],
        out_shape=[
            jax.ShapeDtypeStruct((NN, FD), jnp.float32),
            jax.ShapeDtypeStruct((NN, 1), jnp.float32),
        ],
    )(x, degp, w)


def _tc_layer(t, g, dinv, b, w, relu, emit_h=False):
    def body(t_ref, g_ref, dinv_ref, b_ref, w_ref, og_ref, *oh):
        h = (t_ref[0] + t_ref[1] + g_ref[...]) * dinv_ref[...] + b_ref[...]
        if relu:
            h = jnp.maximum(h, 0.0)
        if emit_h:
            oh[0][...] = h
        og_ref[...] = jnp.dot(h, w_ref[...],
                              preferred_element_type=jnp.float32) * dinv_ref[...]

    out_specs = [pl.BlockSpec((BN, FD), lambda i: (i, 0))]
    out_shape = [jax.ShapeDtypeStruct((NN, FD), jnp.float32)]
    if emit_h:
        out_specs.append(pl.BlockSpec((BN, FD), lambda i: (i, 0)))
        out_shape.append(jax.ShapeDtypeStruct((NN, FD), jnp.float32))
    res = pl.pallas_call(
        body,
        grid=(NN // BN,),
        in_specs=[
            pl.BlockSpec((2, BN, FD), lambda i: (0, i, 0)),
            pl.BlockSpec((BN, FD), lambda i: (i, 0)),
            pl.BlockSpec((BN, 1), lambda i: (i, 0)),
            pl.BlockSpec((1, FD), lambda i: (0, 0)),
            pl.BlockSpec((FD, FD), lambda i: (0, 0)),
        ],
        out_specs=out_specs,
        out_shape=out_shape,
    )(t, g, dinv, b, w)
    return res if emit_h else res[0]


def _tc_epilogue(t, g, dinv, b):
    def body(t_ref, g_ref, dinv_ref, b_ref, o_ref):
        o_ref[...] = ((t_ref[0] + t_ref[1] + g_ref[...]) * dinv_ref[...]
                      + b_ref[...])

    return pl.pallas_call(
        body,
        grid=(NN // BN,),
        in_specs=[
            pl.BlockSpec((2, BN, FD), lambda i: (0, i, 0)),
            pl.BlockSpec((BN, FD), lambda i: (i, 0)),
            pl.BlockSpec((BN, 1), lambda i: (i, 0)),
            pl.BlockSpec((1, FD), lambda i: (0, 0)),
        ],
        out_specs=pl.BlockSpec((BN, FD), lambda i: (i, 0)),
        out_shape=jax.ShapeDtypeStruct((NN, FD), jnp.float32),
    )(t, g, dinv, b)


def _pad_w(w):
    fi, fo = w.shape
    return jnp.pad(w, ((0, FD - fi), (0, FD - fo)))


def _pad_b(b):
    return jnp.pad(b, (0, FD - b.shape[0])).reshape(1, FD)


def kernel(x, edge_index, W1, b1, W2, b2, W3, b3, W4, b4, W5, b5, W6, b6):
    pad = EP - EE
    srcp = jnp.concatenate([edge_index[0],
                            jnp.arange(pad, dtype=jnp.int32) % NN])
    padrows = NN + (jnp.arange(pad, dtype=jnp.int32) % (NP - NN))
    dstp = jnp.concatenate([edge_index[1], padrows])
    src4 = srcp.reshape(NW, NBLK, IB, CH)
    dst4 = dstp.reshape(NW, NBLK, IB, CH)
    dstf = edge_index[1].reshape(NW, EPW)

    degp = _sc_degree(dstf)[:, :, None]
    g1, dinv = _tc_first(x, degp, _pad_w(W1))
    t1 = _sc_propagate(g1, src4, dst4)
    g2 = _tc_layer(t1, g1, dinv, _pad_b(b1), _pad_w(W2), relu=True)
    t2 = _sc_propagate(g2, src4, dst4)
    g3 = _tc_layer(t2, g2, dinv, _pad_b(b2), _pad_w(W3), relu=True)
    t3 = _sc_propagate(g3, src4, dst4)
    g4, z = _tc_layer(t3, g3, dinv, _pad_b(b3), _pad_w(W4),
                      relu=False, emit_h=True)
    t4 = _sc_propagate(g4, src4, dst4)
    g5 = _tc_layer(t4, g4, dinv, _pad_b(b4), _pad_w(W5), relu=True)
    t5 = _sc_propagate(g5, src4, dst4)
    g6 = _tc_layer(t5, g5, dinv, _pad_b(b5), _pad_w(W6), relu=True)
    t6 = _sc_propagate(g6, src4, dst4)
    x_recon = _tc_epilogue(t6, g6, dinv, _pad_b(b6))
    return (x_recon, z[:, :32])
